# baseline (speedup 1.0000x reference)
import math

import jax
import jax.numpy as jnp
from jax import lax
from jax.experimental import pallas as pl

_BLOCK_B = 128


def _fused_attn_kernel(z_ref, ctx_ref, embd_ref, out_ref):
    d = embd_ref.shape[1]
    ctx = ctx_ref[...]
    embd = embd_ref[...]
    scores = lax.dot_general(
        ctx, embd, (((1,), (1,)), ((), ())),
        preferred_element_type=jnp.float32,
    ) * (1.0 / math.sqrt(d))
    mf = (z_ref[...] > 0).astype(jnp.float32)
    row_max = jnp.max(scores, axis=1, keepdims=True)
    ex = jnp.exp(scores - row_max) * mf
    denom = jnp.sum(ex, axis=1, keepdims=True)
    denom = jnp.where(denom == 0.0, 1.0, denom)
    counts = jnp.maximum(jnp.sum(mf, axis=1, keepdims=True), 1.0)
    acc = jnp.dot(ex, embd, preferred_element_type=jnp.float32)
    out_ref[...] = acc / (denom * counts)


def kernel(z_sparse, context_embedding, embd_weight):
    B, F = z_sparse.shape
    d = embd_weight.shape[1]
    grid = (B // _BLOCK_B,)
    return pl.pallas_call(
        _fused_attn_kernel,
        grid=grid,
        in_specs=[
            pl.BlockSpec((_BLOCK_B, F), lambda i: (i, 0)),
            pl.BlockSpec((_BLOCK_B, d), lambda i: (i, 0)),
            pl.BlockSpec((F, d), lambda i: (0, 0)),
        ],
        out_specs=pl.BlockSpec((_BLOCK_B, d), lambda i: (i, 0)),
        out_shape=jax.ShapeDtypeStruct((B, d), jnp.float32),
    )(z_sparse, context_embedding, embd_weight)


# speedup vs baseline: 1.3656x; 1.3656x over previous
import math

import jax
import jax.numpy as jnp
from jax import lax
from jax.experimental import pallas as pl

_BLOCK_B = 512


def _fused_attn_kernel(z_ref, ctx_ref, embd_ref, out_ref):
    d = embd_ref.shape[1]
    ctx = ctx_ref[...]
    embd = embd_ref[...]
    scores = lax.dot_general(
        ctx, embd, (((1,), (1,)), ((), ())),
        preferred_element_type=jnp.float32,
    ) * (1.0 / math.sqrt(d))
    mf = (z_ref[...] > 0).astype(jnp.float32)
    row_max = jnp.max(scores, axis=1, keepdims=True)
    ex = jnp.exp(scores - row_max) * mf
    denom = jnp.sum(ex, axis=1, keepdims=True)
    denom = jnp.where(denom == 0.0, 1.0, denom)
    counts = jnp.maximum(jnp.sum(mf, axis=1, keepdims=True), 1.0)
    acc = jnp.dot(ex, embd, preferred_element_type=jnp.float32)
    out_ref[...] = acc / (denom * counts)


def kernel(z_sparse, context_embedding, embd_weight):
    B, F = z_sparse.shape
    d = embd_weight.shape[1]
    grid = (B // _BLOCK_B,)
    return pl.pallas_call(
        _fused_attn_kernel,
        grid=grid,
        in_specs=[
            pl.BlockSpec((_BLOCK_B, F), lambda i: (i, 0)),
            pl.BlockSpec((_BLOCK_B, d), lambda i: (i, 0)),
            pl.BlockSpec((F, d), lambda i: (0, 0)),
        ],
        out_specs=pl.BlockSpec((_BLOCK_B, d), lambda i: (i, 0)),
        out_shape=jax.ShapeDtypeStruct((B, d), jnp.float32),
    )(z_sparse, context_embedding, embd_weight)
